# baseline (speedup 1.0000x reference)
import jax
import jax.numpy as jnp
from jax import lax
from jax.experimental import pallas as pl
from jax.experimental.pallas import tpu as pltpu
from jax.experimental.pallas import tpu_sc as plsc

N = 10000
E = 320000
D = 128
B = 100

DH = D // 2
NPAD = 10240
RT = NPAD // 16
RC = 128
EC = 128
EPAD = 321536
ETP = EPAD // 16
NCH_E = ETP // EC
NCH_R = RT // RC


def _fast_rsqrt(d):
    xi = lax.bitcast_convert_type(d, jnp.int32)
    xi = jnp.int32(0x5F3759DF) - (xi >> 1)
    r = lax.bitcast_convert_type(xi, jnp.float32)
    r = r * (1.5 - 0.5 * d * r * r)
    r = r * (1.5 - 0.5 * d * r * r)
    r = r * (1.5 - 0.5 * d * r * r)
    return r


def _sc_gcn_body(xw0, xw1, eir, bs, outT, outS, qout, y0, y1, acc, degsp, qacc,
                 sidx, didx, rows0, rows1, rows2, rows3, ones, dinv,
                 rbufA, rbufB, bvec,
                 semG0, semG1, semG2, semG3, semS0, semS1, semS2, semS3,
                 semD, semW, semI):
    c = lax.axis_index("c")
    s = lax.axis_index("s")
    boff = c * NPAD
    rows = (rows0, rows1, rows2, rows3)
    semG = (semG0, semG1, semG2, semG3)
    semS = (semS0, semS1, semS2, semS3)

    def _zero_body(r, _):
        dinv[r, :] = jnp.zeros((16,), jnp.float32)
        return _
    lax.fori_loop(0, RT, _zero_body, None)

    def _ones_body(r, _):
        ones[r, :] = jnp.ones((16,), jnp.float32)
        return _
    lax.fori_loop(0, EC, _ones_body, None)

    pltpu.sync_copy(dinv, degsp.at[pl.ds(RT * s, RT)])

    @pl.when((c == 0) & (s == 0))
    def _():
        def _zq_body(r, _):
            for k in range(DH // 16):
                rows0[r, pl.ds(k * 16, 16)] = jnp.zeros((16,), jnp.float32)
            return _
        lax.fori_loop(0, B + 4, _zq_body, None)
        pltpu.sync_copy(rows0.at[pl.ds(0, B + 4)], qacc)
    plsc.subcore_barrier()

    def _didx_load(j, sem):
        pltpu.async_copy(eir.at[c, 1, s, j], didx.at[j % 8], sem)

    def _didx_wait(j, sem):
        pltpu.make_async_copy(eir.at[c, 1, s, j], didx.at[j % 8], sem).wait()

    for j in range(4):
        _didx_load(j, semI)
    for j in range(4):
        _didx_wait(j, semI)
    for j in range(4, 8):
        _didx_load(j, semI)

    def _deg_body(q, _):
        j0 = 4 * q
        for b in range(4):
            pltpu.async_copy(ones, degsp.at[didx.at[(j0 + b) % 8]], semD,
                             add=True)
        for b in range(4):
            @pl.when(j0 + 4 + b < NCH_E)
            def _(b=b):
                _didx_wait(j0 + 4 + b, semI)
        for b in range(4):
            pltpu.make_async_copy(ones, degsp.at[didx.at[(j0 + b) % 8]],
                                  semD).wait()
        for b in range(4):
            @pl.when(j0 + 8 + b < NCH_E)
            def _(b=b):
                _didx_load(j0 + 8 + b, semI)
        return _
    lax.fori_loop(0, (NCH_E - 1) // 4, _deg_body, None)
    jl = NCH_E - 1
    pltpu.sync_copy(ones, degsp.at[didx.at[jl % 8]], add=True)
    plsc.subcore_barrier()

    pltpu.sync_copy(degsp.at[pl.ds(RT * s, RT)], dinv)

    def _rs_body(r, _):
        d = dinv[r, :] + 1.0
        dinv[r, :] = _fast_rsqrt(d)
        return _
    lax.fori_loop(0, RT, _rs_body, None)

    rbase = s * RT
    bufs = (rbufA, rbufB)
    for p, xwp, yp in ((0, xw0, y0), (1, xw1, y1)):
        pltpu.sync_copy(bs.at[c, pl.ds(p * DH, DH)], bvec)

        pltpu.async_copy(xwp.at[c, pl.ds(rbase, RC)], rbufA, semG0)
        for rj in range(NCH_R):
            rb = bufs[rj % 2]
            row0 = rbase + rj * RC
            pltpu.make_async_copy(xwp.at[c, pl.ds(row0, RC)], rb, semG0).wait()
            if rj + 1 < NCH_R:
                pltpu.async_copy(
                    xwp.at[c, pl.ds(row0 + RC, RC)], bufs[(rj + 1) % 2], semG0)

            def _scale_body(r, _, rj=rj, rb=rb):
                sv = dinv[rj * RC + r, :]
                for k in range(DH // 16):
                    rb[r, pl.ds(k * 16, 16)] = rb[r, pl.ds(k * 16, 16)] * sv
                return _
            lax.fori_loop(0, RC, _scale_body, None)
            pltpu.async_copy(rb, yp.at[pl.ds(c * NPAD + row0, RC)], semW)
            pltpu.sync_copy(rb, acc.at[pl.ds(row0, RC)])
            pltpu.make_async_copy(rb, yp.at[pl.ds(c * NPAD + row0, RC)], semW).wait()
        plsc.subcore_barrier()

        def _sidx_load(j, sem, yp=yp):
            pltpu.async_copy(eir.at[c, 0, s, j], sidx.at[j % 8], sem)

        def _idx_wait(j, sem, yp=yp):
            pltpu.make_async_copy(eir.at[c, 0, s, j], sidx.at[j % 8], sem).wait()
            pltpu.make_async_copy(eir.at[c, 1, s, j], didx.at[j % 8], sem).wait()

        def _gather(j, rb, sem, yp=yp):
            for k in range(EC // 16):
                sidx[j % 8, pl.ds(k * 16, 16)] = (
                    sidx[j % 8, pl.ds(k * 16, 16)] + boff)
            pltpu.async_copy(yp.at[sidx.at[j % 8]], rb, sem)

        def _gather_wait(j, rb, sem, yp=yp):
            pltpu.make_async_copy(yp.at[sidx.at[j % 8]], rb, sem).wait()

        def _scat(j, rb, sem):
            pltpu.async_copy(rb, acc.at[didx.at[j % 8]], sem, add=True)

        def _scat_wait(j, rb, sem):
            pltpu.make_async_copy(rb, acc.at[didx.at[j % 8]], sem).wait()

        for j in range(4):
            _sidx_load(j, semI)
            _didx_load(j, semI)
        for j in range(4):
            _idx_wait(j, semI)
        for b in range(4):
            _gather(b, rows[b], semG[b])
        for j in range(4, 8):
            _sidx_load(j, semI)
            _didx_load(j, semI)

        def _edge_body(q, _, yp=yp):
            j0 = 4 * q
            for b in range(4):
                _gather_wait(j0 + b, rows[b], semG[b])
                _scat(j0 + b, rows[b], semS[b])
            for b in range(4):
                @pl.when(j0 + 4 + b < NCH_E)
                def _(b=b):
                    _idx_wait(j0 + 4 + b, semI)
            for b in range(4):
                _scat_wait(j0 + b, rows[b], semS[b])

                @pl.when(j0 + 4 + b < NCH_E)
                def _(b=b):
                    _gather(j0 + 4 + b, rows[b], semG[b])
            for b in range(4):
                @pl.when(j0 + 8 + b < NCH_E)
                def _(b=b):
                    _sidx_load(j0 + 8 + b, semI)
                    _didx_load(j0 + 8 + b, semI)
            return _
        lax.fori_loop(0, (NCH_E - 1) // 4, _edge_body, None)
        jl = NCH_E - 1
        _gather_wait(jl, rows[0], semG[0])
        pltpu.sync_copy(rows[0], acc.at[didx.at[jl % 8]], add=True)
        plsc.subcore_barrier()

        pltpu.async_copy(acc.at[pl.ds(rbase, RC)], rbufA, semG0)
        for rj in range(NCH_R):
            rb = bufs[rj % 2]
            row0 = rbase + rj * RC
            pltpu.make_async_copy(acc.at[pl.ds(row0, RC)], rb, semG0).wait()
            if rj + 1 < NCH_R:
                pltpu.async_copy(
                    acc.at[pl.ds(row0 + RC, RC)], bufs[(rj + 1) % 2], semG0)

            def _out_body(r, _, rj=rj, rb=rb):
                sv = dinv[rj * RC + r, :]
                for k in range(DH // 16):
                    rb[r, pl.ds(k * 16, 16)] = (
                        rb[r, pl.ds(k * 16, 16)] * sv + bvec[pl.ds(k * 16, 16)])
                return _
            lax.fori_loop(0, RC, _out_body, None)
            for cc, outp in ((0, outT), (1, outS)):
                @pl.when((c == cc) & (row0 + RC <= N))
                def _(outp=outp, row0=row0, rb=rb, p=p):
                    pltpu.sync_copy(
                        rb, outp.at[pl.ds(row0, RC), pl.ds(p * DH, DH)])

                @pl.when((c == cc) & (row0 < N) & (row0 + RC > N))
                def _(outp=outp, row0=row0, rb=rb, p=p):
                    pltpu.sync_copy(
                        rb.at[pl.ds(0, N % RC)],
                        outp.at[pl.ds(row0, N % RC), pl.ds(p * DH, DH)])

            @pl.when(c == 0)
            def _(row0=row0, rb=rb):
                for k in range(EC // 16):
                    rv = row0 + k * 16 + lax.iota(jnp.int32, 16)
                    didx[0, pl.ds(k * 16, 16)] = (rv * 5243) >> 19

                def _q_body(r, _):
                    for k in range(DH // 16):
                        rows1[r, pl.ds(k * 16, 16)] = (
                            rb[r, pl.ds(k * 16, 16)] * (1.0 / (N // B)))
                    return _
                lax.fori_loop(0, RC, _q_body, None)
                pltpu.sync_copy(rows1, qacc.at[didx.at[0]], add=True)
        plsc.subcore_barrier()

        @pl.when((c == 0) & (s == 0))
        def _(p=p):
            pltpu.sync_copy(qacc, rows1.at[pl.ds(0, B + 4)])
            pltpu.sync_copy(rows1.at[pl.ds(0, B + 4)],
                            qout.at[pl.ds(0, B + 4), pl.ds(p * DH, DH)])
            if p == 0:
                def _zq_body(r, _):
                    for k in range(DH // 16):
                        rows1[r, pl.ds(k * 16, 16)] = jnp.zeros(
                            (16,), jnp.float32)
                    return _
                lax.fori_loop(0, B + 4, _zq_body, None)
                pltpu.sync_copy(rows1.at[pl.ds(0, B + 4)], qacc)


@jax.jit
def _sc_gcn(xw0, xw1, eir, bs):
    mesh = plsc.VectorSubcoreMesh(core_axis_name="c", subcore_axis_name="s")
    f = pl.kernel(
        _sc_gcn_body,
        out_type=[jax.ShapeDtypeStruct((N, D), jnp.float32),
                  jax.ShapeDtypeStruct((N, D), jnp.float32),
                  jax.ShapeDtypeStruct((B + 4, D), jnp.float32),
                  jax.ShapeDtypeStruct((2 * NPAD, DH), jnp.float32),
                  jax.ShapeDtypeStruct((2 * NPAD, DH), jnp.float32)],
        mesh=mesh,
        compiler_params=pltpu.CompilerParams(use_tc_tiling_on_sc=False),
        scratch_types=[
            pltpu.VMEM_SHARED((NPAD, DH), jnp.float32),
            pltpu.VMEM_SHARED((NPAD, 16), jnp.float32),
            pltpu.VMEM_SHARED((B + 4, DH), jnp.float32),
            pltpu.VMEM((8, EC), jnp.int32),
            pltpu.VMEM((8, EC), jnp.int32),
            pltpu.VMEM((EC, DH), jnp.float32),
            pltpu.VMEM((EC, DH), jnp.float32),
            pltpu.VMEM((EC, DH), jnp.float32),
            pltpu.VMEM((EC, DH), jnp.float32),
            pltpu.VMEM((EC, 16), jnp.float32),
            pltpu.VMEM((RT, 16), jnp.float32),
            pltpu.VMEM((RC, DH), jnp.float32),
            pltpu.VMEM((RC, DH), jnp.float32),
            pltpu.VMEM((DH,), jnp.float32),
            pltpu.SemaphoreType.DMA,
            pltpu.SemaphoreType.DMA,
            pltpu.SemaphoreType.DMA,
            pltpu.SemaphoreType.DMA,
            pltpu.SemaphoreType.DMA,
            pltpu.SemaphoreType.DMA,
            pltpu.SemaphoreType.DMA,
            pltpu.SemaphoreType.DMA,
            pltpu.SemaphoreType.DMA,
            pltpu.SemaphoreType.DMA,
            pltpu.SemaphoreType.DMA,
        ],
    )
    return f(xw0, xw1, eir, bs)


def _mm_body(x_ref, w_ref, o0_ref, o1_ref):
    r = jnp.dot(x_ref[0], w_ref[0], preferred_element_type=jnp.float32)
    o0_ref[0] = r[:, :DH]
    o1_ref[0] = r[:, DH:]


@jax.jit
def _tc_matmul(xs, Ws):
    BM = 1280
    return pl.pallas_call(
        _mm_body,
        grid=(2, NPAD // BM),
        in_specs=[pl.BlockSpec((1, BM, D), lambda b, i: (b, i, 0)),
                  pl.BlockSpec((1, D, D), lambda b, i: (b, 0, 0))],
        out_specs=[pl.BlockSpec((1, BM, DH), lambda b, i: (b, i, 0)),
                   pl.BlockSpec((1, BM, DH), lambda b, i: (b, i, 0))],
        out_shape=[jax.ShapeDtypeStruct((2, NPAD, DH), jnp.float32),
                   jax.ShapeDtypeStruct((2, NPAD, DH), jnp.float32)],
    )(xs, Ws)


def kernel(text_x, text_edge_index, text_batch, scene_x, scene_edge_index,
           scene_batch, W_text, b_text, W_scene, b_scene):
    Ws = jnp.stack([W_text, W_scene])
    bs = jnp.stack([b_text, b_scene])
    ei = jnp.stack([text_edge_index, scene_edge_index])
    ei_p = jnp.concatenate(
        [ei, jnp.full((2, 2, EPAD - E), NPAD - 1, ei.dtype)], axis=-1)
    eir = ei_p.reshape(2, 2, 16, NCH_E, EC)

    xs = jnp.stack([text_x, scene_x])
    xw0, xw1 = _tc_matmul(xs, Ws)
    out_t, out_s, qpad, _, _ = _sc_gcn(xw0, xw1, eir, bs)
    question = qpad[:B]

    contextual_words = out_t.reshape(B, N // B, D)
    scene_graph_feats = out_s.reshape(B, N // B, D)
    return contextual_words, question, scene_graph_feats


# speedup vs baseline: 1.0139x; 1.0139x over previous
import jax
import jax.numpy as jnp
from jax import lax
from jax.experimental import pallas as pl
from jax.experimental.pallas import tpu as pltpu
from jax.experimental.pallas import tpu_sc as plsc

N = 10000
E = 320000
D = 128
B = 100

DH = D // 2
NPAD = 10240
RT = NPAD // 16
RC = 128
EC = 128
EPAD = 321536
ETP = EPAD // 16
NCH_E = ETP // EC
NCH_R = RT // RC


def _fast_rsqrt(d):
    xi = lax.bitcast_convert_type(d, jnp.int32)
    xi = jnp.int32(0x5F3759DF) - (xi >> 1)
    r = lax.bitcast_convert_type(xi, jnp.float32)
    r = r * (1.5 - 0.5 * d * r * r)
    r = r * (1.5 - 0.5 * d * r * r)
    r = r * (1.5 - 0.5 * d * r * r)
    return r


def _sc_gcn_body(xw0, xw1, eir, bs, outT, outS, qout, y0, y1, acc, degsp, qacc,
                 sidx, didx, rows0, rows1, rows2, rows3, rows4, ones,
                 dinv, rbufA, rbufB, bvec,
                 semG0, semG1, semG2, semG3, semG4,
                 semS0, semS1, semS2, semS3, semS4,
                 semD, semW, semI):
    c = lax.axis_index("c")
    s = lax.axis_index("s")
    boff = c * NPAD
    rows = (rows0, rows1, rows2, rows3, rows4)
    semG = (semG0, semG1, semG2, semG3, semG4)
    semS = (semS0, semS1, semS2, semS3, semS4)

    def _zero_body(r, _):
        dinv[r, :] = jnp.zeros((16,), jnp.float32)
        return _
    lax.fori_loop(0, RT, _zero_body, None)

    def _ones_body(r, _):
        ones[r, :] = jnp.ones((16,), jnp.float32)
        return _
    lax.fori_loop(0, EC, _ones_body, None)

    pltpu.sync_copy(dinv, degsp.at[pl.ds(RT * s, RT)])

    @pl.when((c == 0) & (s == 0))
    def _():
        def _zq_body(r, _):
            for k in range(DH // 16):
                rows0[r, pl.ds(k * 16, 16)] = jnp.zeros((16,), jnp.float32)
            return _
        lax.fori_loop(0, B + 4, _zq_body, None)
        pltpu.sync_copy(rows0.at[pl.ds(0, B + 4)], qacc)
    plsc.subcore_barrier()

    def _didx_load(j, sem):
        pltpu.async_copy(eir.at[c, 1, s, j], didx.at[j % 8], sem)

    def _didx_wait(j, sem):
        pltpu.make_async_copy(eir.at[c, 1, s, j], didx.at[j % 8], sem).wait()

    for j in range(4):
        _didx_load(j, semI)
    for j in range(4):
        _didx_wait(j, semI)
    for j in range(4, 8):
        _didx_load(j, semI)

    def _deg_body(q, _):
        j0 = 4 * q
        for b in range(4):
            pltpu.async_copy(ones, degsp.at[didx.at[(j0 + b) % 8]], semD,
                             add=True)
        for b in range(4):
            @pl.when(j0 + 4 + b < NCH_E)
            def _(b=b):
                _didx_wait(j0 + 4 + b, semI)
        for b in range(4):
            pltpu.make_async_copy(ones, degsp.at[didx.at[(j0 + b) % 8]],
                                  semD).wait()
        for b in range(4):
            @pl.when(j0 + 8 + b < NCH_E)
            def _(b=b):
                _didx_load(j0 + 8 + b, semI)
        return _
    lax.fori_loop(0, (NCH_E - 1) // 4, _deg_body, None)
    jl = NCH_E - 1
    pltpu.sync_copy(ones, degsp.at[didx.at[jl % 8]], add=True)
    plsc.subcore_barrier()

    pltpu.sync_copy(degsp.at[pl.ds(RT * s, RT)], dinv)

    def _rs_body(r, _):
        d = dinv[r, :] + 1.0
        dinv[r, :] = _fast_rsqrt(d)
        return _
    lax.fori_loop(0, RT, _rs_body, None)

    rbase = s * RT
    bufs = (rbufA, rbufB)
    for p, xwp, yp in ((0, xw0, y0), (1, xw1, y1)):
        pltpu.sync_copy(bs.at[c, pl.ds(p * DH, DH)], bvec)

        pltpu.async_copy(xwp.at[c, pl.ds(rbase, RC)], rbufA, semG0)
        for rj in range(NCH_R):
            rb = bufs[rj % 2]
            row0 = rbase + rj * RC
            pltpu.make_async_copy(xwp.at[c, pl.ds(row0, RC)], rb, semG0).wait()
            if rj + 1 < NCH_R:
                pltpu.async_copy(
                    xwp.at[c, pl.ds(row0 + RC, RC)], bufs[(rj + 1) % 2], semG0)

            def _scale_body(r, _, rj=rj, rb=rb):
                sv = dinv[rj * RC + r, :]
                for k in range(DH // 16):
                    rb[r, pl.ds(k * 16, 16)] = rb[r, pl.ds(k * 16, 16)] * sv
                return _
            lax.fori_loop(0, RC, _scale_body, None)
            pltpu.async_copy(rb, yp.at[pl.ds(c * NPAD + row0, RC)], semW)
            pltpu.sync_copy(rb, acc.at[pl.ds(row0, RC)])
            pltpu.make_async_copy(rb, yp.at[pl.ds(c * NPAD + row0, RC)], semW).wait()
        plsc.subcore_barrier()

        def _eidx_load(j, sem, yp=yp):
            pltpu.async_copy(eir.at[c, 0, s, j], sidx.at[j % 10], sem)
            pltpu.async_copy(eir.at[c, 1, s, j], didx.at[j % 10], sem)

        def _idx_wait(j, sem, yp=yp):
            pltpu.make_async_copy(eir.at[c, 0, s, j], sidx.at[j % 10], sem).wait()
            pltpu.make_async_copy(eir.at[c, 1, s, j], didx.at[j % 10], sem).wait()

        def _gather(j, rb, sem, yp=yp):
            for k in range(EC // 16):
                sidx[j % 10, pl.ds(k * 16, 16)] = (
                    sidx[j % 10, pl.ds(k * 16, 16)] + boff)
            pltpu.async_copy(yp.at[sidx.at[j % 10]], rb, sem)

        def _gather_wait(j, rb, sem, yp=yp):
            pltpu.make_async_copy(yp.at[sidx.at[j % 10]], rb, sem).wait()

        def _scat(j, rb, sem):
            pltpu.async_copy(rb, acc.at[didx.at[j % 10]], sem, add=True)

        def _scat_wait(j, rb, sem):
            pltpu.make_async_copy(rb, acc.at[didx.at[j % 10]], sem).wait()

        for j in range(5):
            _eidx_load(j, semI)
        for j in range(5):
            _idx_wait(j, semI)
        for b in range(5):
            _gather(b, rows[b], semG[b])
        for j in range(5, 10):
            _eidx_load(j, semI)

        def _edge_body(q, _, yp=yp):
            j0 = 5 * q
            for b in range(5):
                _gather_wait(j0 + b, rows[b], semG[b])
                _scat(j0 + b, rows[b], semS[b])
            for b in range(5):
                @pl.when(j0 + 5 + b < NCH_E)
                def _(b=b):
                    _idx_wait(j0 + 5 + b, semI)
            for b in range(5):
                _scat_wait(j0 + b, rows[b], semS[b])

                @pl.when(j0 + 5 + b < NCH_E)
                def _(b=b):
                    _gather(j0 + 5 + b, rows[b], semG[b])
            for b in range(5):
                @pl.when(j0 + 10 + b < NCH_E)
                def _(b=b):
                    _eidx_load(j0 + 10 + b, semI)
            return _
        lax.fori_loop(0, (NCH_E - 2) // 5, _edge_body, None)
        jl = NCH_E - 1
        _gather_wait(jl, rows[0], semG[0])
        pltpu.sync_copy(rows[0], acc.at[didx.at[jl % 10]], add=True)
        plsc.subcore_barrier()

        pltpu.async_copy(acc.at[pl.ds(rbase, RC)], rbufA, semG0)
        for rj in range(NCH_R):
            rb = bufs[rj % 2]
            row0 = rbase + rj * RC
            pltpu.make_async_copy(acc.at[pl.ds(row0, RC)], rb, semG0).wait()
            if rj + 1 < NCH_R:
                pltpu.async_copy(
                    acc.at[pl.ds(row0 + RC, RC)], bufs[(rj + 1) % 2], semG0)

            def _out_body(r, _, rj=rj, rb=rb):
                sv = dinv[rj * RC + r, :]
                for k in range(DH // 16):
                    rb[r, pl.ds(k * 16, 16)] = (
                        rb[r, pl.ds(k * 16, 16)] * sv + bvec[pl.ds(k * 16, 16)])
                return _
            lax.fori_loop(0, RC, _out_body, None)
            for cc, outp in ((0, outT), (1, outS)):
                @pl.when((c == cc) & (row0 + RC <= N))
                def _(outp=outp, row0=row0, rb=rb, p=p):
                    pltpu.sync_copy(
                        rb, outp.at[pl.ds(row0, RC), pl.ds(p * DH, DH)])

                @pl.when((c == cc) & (row0 < N) & (row0 + RC > N))
                def _(outp=outp, row0=row0, rb=rb, p=p):
                    pltpu.sync_copy(
                        rb.at[pl.ds(0, N % RC)],
                        outp.at[pl.ds(row0, N % RC), pl.ds(p * DH, DH)])

            @pl.when(c == 0)
            def _(row0=row0, rb=rb):
                for k in range(EC // 16):
                    rv = row0 + k * 16 + lax.iota(jnp.int32, 16)
                    didx[0, pl.ds(k * 16, 16)] = (rv * 5243) >> 19

                def _q_body(r, _):
                    for k in range(DH // 16):
                        rows1[r, pl.ds(k * 16, 16)] = (
                            rb[r, pl.ds(k * 16, 16)] * (1.0 / (N // B)))
                    return _
                lax.fori_loop(0, RC, _q_body, None)
                pltpu.sync_copy(rows1, qacc.at[didx.at[0]], add=True)
        plsc.subcore_barrier()

        @pl.when((c == 0) & (s == 0))
        def _(p=p):
            pltpu.sync_copy(qacc, rows1.at[pl.ds(0, B + 4)])
            pltpu.sync_copy(rows1.at[pl.ds(0, B + 4)],
                            qout.at[pl.ds(0, B + 4), pl.ds(p * DH, DH)])
            if p == 0:
                def _zq_body(r, _):
                    for k in range(DH // 16):
                        rows1[r, pl.ds(k * 16, 16)] = jnp.zeros(
                            (16,), jnp.float32)
                    return _
                lax.fori_loop(0, B + 4, _zq_body, None)
                pltpu.sync_copy(rows1.at[pl.ds(0, B + 4)], qacc)


@jax.jit
def _sc_gcn(xw0, xw1, eir, bs):
    mesh = plsc.VectorSubcoreMesh(core_axis_name="c", subcore_axis_name="s")
    f = pl.kernel(
        _sc_gcn_body,
        out_type=[jax.ShapeDtypeStruct((N, D), jnp.float32),
                  jax.ShapeDtypeStruct((N, D), jnp.float32),
                  jax.ShapeDtypeStruct((B + 4, D), jnp.float32),
                  jax.ShapeDtypeStruct((2 * NPAD, DH), jnp.float32),
                  jax.ShapeDtypeStruct((2 * NPAD, DH), jnp.float32)],
        mesh=mesh,
        compiler_params=pltpu.CompilerParams(use_tc_tiling_on_sc=False),
        scratch_types=[
            pltpu.VMEM_SHARED((NPAD, DH), jnp.float32),
            pltpu.VMEM_SHARED((NPAD, 16), jnp.float32),
            pltpu.VMEM_SHARED((B + 4, DH), jnp.float32),
            pltpu.VMEM((10, EC), jnp.int32),
            pltpu.VMEM((10, EC), jnp.int32),
            pltpu.VMEM((EC, DH), jnp.float32),
            pltpu.VMEM((EC, DH), jnp.float32),
            pltpu.VMEM((EC, DH), jnp.float32),
            pltpu.VMEM((EC, DH), jnp.float32),
            pltpu.VMEM((EC, DH), jnp.float32),
            pltpu.VMEM((EC, 16), jnp.float32),
            pltpu.VMEM((RT, 16), jnp.float32),
            pltpu.VMEM((RC, DH), jnp.float32),
            pltpu.VMEM((RC, DH), jnp.float32),
            pltpu.VMEM((DH,), jnp.float32),
            pltpu.SemaphoreType.DMA,
            pltpu.SemaphoreType.DMA,
            pltpu.SemaphoreType.DMA,
            pltpu.SemaphoreType.DMA,
            pltpu.SemaphoreType.DMA,
            pltpu.SemaphoreType.DMA,
            pltpu.SemaphoreType.DMA,
            pltpu.SemaphoreType.DMA,
            pltpu.SemaphoreType.DMA,
            pltpu.SemaphoreType.DMA,
            pltpu.SemaphoreType.DMA,
            pltpu.SemaphoreType.DMA,
            pltpu.SemaphoreType.DMA,
        ],
    )
    return f(xw0, xw1, eir, bs)


def _mm_body(x_ref, w_ref, o0_ref, o1_ref):
    r = jnp.dot(x_ref[0], w_ref[0], preferred_element_type=jnp.float32)
    o0_ref[0] = r[:, :DH]
    o1_ref[0] = r[:, DH:]


@jax.jit
def _tc_matmul(xs, Ws):
    BM = 1280
    return pl.pallas_call(
        _mm_body,
        grid=(2, NPAD // BM),
        in_specs=[pl.BlockSpec((1, BM, D), lambda b, i: (b, i, 0)),
                  pl.BlockSpec((1, D, D), lambda b, i: (b, 0, 0))],
        out_specs=[pl.BlockSpec((1, BM, DH), lambda b, i: (b, i, 0)),
                   pl.BlockSpec((1, BM, DH), lambda b, i: (b, i, 0))],
        out_shape=[jax.ShapeDtypeStruct((2, NPAD, DH), jnp.float32),
                   jax.ShapeDtypeStruct((2, NPAD, DH), jnp.float32)],
    )(xs, Ws)


def kernel(text_x, text_edge_index, text_batch, scene_x, scene_edge_index,
           scene_batch, W_text, b_text, W_scene, b_scene):
    Ws = jnp.stack([W_text, W_scene])
    bs = jnp.stack([b_text, b_scene])
    ei = jnp.stack([text_edge_index, scene_edge_index])
    ei_p = jnp.concatenate(
        [ei, jnp.full((2, 2, EPAD - E), NPAD - 1, ei.dtype)], axis=-1)
    eir = ei_p.reshape(2, 2, 16, NCH_E, EC)

    xs = jnp.stack([text_x, scene_x])
    xw0, xw1 = _tc_matmul(xs, Ws)
    out_t, out_s, qpad, _, _ = _sc_gcn(xw0, xw1, eir, bs)
    question = qpad[:B]

    contextual_words = out_t.reshape(B, N // B, D)
    scene_graph_feats = out_s.reshape(B, N // B, D)
    return contextual_words, question, scene_graph_feats


# speedup vs baseline: 1.0153x; 1.0014x over previous
import jax
import jax.numpy as jnp
from jax import lax
from jax.experimental import pallas as pl
from jax.experimental.pallas import tpu as pltpu
from jax.experimental.pallas import tpu_sc as plsc

N = 10000
E = 320000
D = 128
B = 100

DH = D // 2
NPAD = 10240
RT = NPAD // 16
RC = 128
EC = 128
EPAD = 321536
ETP = EPAD // 16
NCH_E = ETP // EC
NCH_R = RT // RC


def _fast_rsqrt(d):
    xi = lax.bitcast_convert_type(d, jnp.int32)
    xi = jnp.int32(0x5F3759DF) - (xi >> 1)
    r = lax.bitcast_convert_type(xi, jnp.float32)
    r = r * (1.5 - 0.5 * d * r * r)
    r = r * (1.5 - 0.5 * d * r * r)
    r = r * (1.5 - 0.5 * d * r * r)
    return r


def _sc_gcn_body(xw0, xw1, eir, bs, outT, outS, qout, y0, y1, acc, degsp, qacc,
                 sidx, didx, rows0, rows1, rows2, rows3, rows4, ones,
                 dinv, rbufA, rbufB, bvec,
                 semG0, semG1, semG2, semG3, semG4,
                 semS0, semS1, semS2, semS3, semS4,
                 semD, semW, semI):
    c = lax.axis_index("c")
    s = lax.axis_index("s")
    boff = c * NPAD
    rows = (rows0, rows1, rows2, rows3, rows4)
    semG = (semG0, semG1, semG2, semG3, semG4)
    semS = (semS0, semS1, semS2, semS3, semS4)

    def _zero_body(r, _):
        dinv[r, :] = jnp.zeros((16,), jnp.float32)
        return _
    lax.fori_loop(0, RT, _zero_body, None)

    def _ones_body(r, _):
        ones[r, :] = jnp.ones((16,), jnp.float32)
        return _
    lax.fori_loop(0, EC, _ones_body, None)

    pltpu.sync_copy(dinv, degsp.at[pl.ds(RT * s, RT)])

    @pl.when((c == 0) & (s == 0))
    def _():
        def _zq_body(r, _):
            for k in range(DH // 16):
                rows0[r, pl.ds(k * 16, 16)] = jnp.zeros((16,), jnp.float32)
            return _
        lax.fori_loop(0, B + 4, _zq_body, None)
        pltpu.sync_copy(rows0.at[pl.ds(0, B + 4)], qacc)
    plsc.subcore_barrier()

    def _didx_load(j, sem):
        pltpu.async_copy(eir.at[c, 1, s, j], didx.at[j % 8], sem)

    def _didx_wait(j, sem):
        pltpu.make_async_copy(eir.at[c, 1, s, j], didx.at[j % 8], sem).wait()

    for j in range(4):
        _didx_load(j, semI)
    for j in range(4):
        _didx_wait(j, semI)
    for j in range(4, 8):
        _didx_load(j, semI)

    def _deg_body(q, _):
        j0 = 4 * q
        for b in range(4):
            pltpu.async_copy(ones, degsp.at[didx.at[(j0 + b) % 8]], semD,
                             add=True)
        for b in range(4):
            @pl.when(j0 + 4 + b < NCH_E)
            def _(b=b):
                _didx_wait(j0 + 4 + b, semI)
        for b in range(4):
            pltpu.make_async_copy(ones, degsp.at[didx.at[(j0 + b) % 8]],
                                  semD).wait()
        for b in range(4):
            @pl.when(j0 + 8 + b < NCH_E)
            def _(b=b):
                _didx_load(j0 + 8 + b, semI)
        return _
    lax.fori_loop(0, (NCH_E - 1) // 4, _deg_body, None)
    jl = NCH_E - 1
    pltpu.sync_copy(ones, degsp.at[didx.at[jl % 8]], add=True)
    plsc.subcore_barrier()

    pltpu.sync_copy(degsp.at[pl.ds(RT * s, RT)], dinv)

    def _rs_body(r, _):
        d = dinv[r, :] + 1.0
        dinv[r, :] = _fast_rsqrt(d)
        return _
    lax.fori_loop(0, RT, _rs_body, None)

    rbase = s * RT
    bufs = (rbufA, rbufB)
    for p, xwp, yp in ((0, xw0, y0), (1, xw1, y1)):
        pltpu.sync_copy(bs.at[c, pl.ds(p * DH, DH)], bvec)

        pltpu.async_copy(xwp.at[c, pl.ds(rbase, RC)], rbufA, semG0)
        for rj in range(NCH_R):
            rb = bufs[rj % 2]
            row0 = rbase + rj * RC
            pltpu.make_async_copy(xwp.at[c, pl.ds(row0, RC)], rb, semG0).wait()
            if rj + 1 < NCH_R:
                pltpu.async_copy(
                    xwp.at[c, pl.ds(row0 + RC, RC)], bufs[(rj + 1) % 2], semG0)

            def _scale_body(r, _, rj=rj, rb=rb):
                sv = dinv[rj * RC + r, :]
                for k in range(DH // 16):
                    rb[r, pl.ds(k * 16, 16)] = rb[r, pl.ds(k * 16, 16)] * sv
                return _
            lax.fori_loop(0, RC, _scale_body, None)
            pltpu.async_copy(rb, yp.at[pl.ds(c * NPAD + row0, RC)], semW)
            pltpu.sync_copy(rb, acc.at[pl.ds(row0, RC)])
            pltpu.make_async_copy(rb, yp.at[pl.ds(c * NPAD + row0, RC)], semW).wait()
        plsc.subcore_barrier()

        def _eidx_load(j, sem, yp=yp):
            pltpu.async_copy(eir.at[c, 0, s, j], sidx.at[j % 10], sem)
            pltpu.async_copy(eir.at[c, 1, s, j], didx.at[j % 10], sem)

        def _idx_wait(j, sem, yp=yp):
            pltpu.make_async_copy(eir.at[c, 0, s, j], sidx.at[j % 10], sem).wait()
            pltpu.make_async_copy(eir.at[c, 1, s, j], didx.at[j % 10], sem).wait()

        def _gather(j, rb, sem, yp=yp):
            for k in range(EC // 16):
                sidx[j % 10, pl.ds(k * 16, 16)] = (
                    sidx[j % 10, pl.ds(k * 16, 16)] + boff)
            pltpu.async_copy(yp.at[sidx.at[j % 10]], rb, sem)

        def _gather_wait(j, rb, sem, yp=yp):
            pltpu.make_async_copy(yp.at[sidx.at[j % 10]], rb, sem).wait()

        def _scat(j, rb, sem):
            pltpu.async_copy(rb, acc.at[didx.at[j % 10]], sem, add=True)

        def _scat_wait(j, rb, sem):
            pltpu.make_async_copy(rb, acc.at[didx.at[j % 10]], sem).wait()

        for j in range(5):
            _eidx_load(j, semI)
        for j in range(5):
            _idx_wait(j, semI)
        for b in range(5):
            _gather(b, rows[b], semG[b])
        for j in range(5, 10):
            _eidx_load(j, semI)

        def _edge_body(q, _, yp=yp):
            j0 = 5 * q
            for b in range(5):
                _gather_wait(j0 + b, rows[b], semG[b])
                _scat(j0 + b, rows[b], semS[b])
            for b in range(5):
                @pl.when(j0 + 5 + b < NCH_E)
                def _(b=b):
                    _idx_wait(j0 + 5 + b, semI)
            for b in range(5):
                _scat_wait(j0 + b, rows[b], semS[b])

                @pl.when(j0 + 5 + b < NCH_E)
                def _(b=b):
                    _gather(j0 + 5 + b, rows[b], semG[b])
            for b in range(5):
                @pl.when(j0 + 10 + b < NCH_E)
                def _(b=b):
                    _eidx_load(j0 + 10 + b, semI)
            return _
        lax.fori_loop(0, (NCH_E - 2) // 5, _edge_body, None)
        for jl in (NCH_E - 2, NCH_E - 1):
            _gather_wait(jl, rows[jl % 5], semG[jl % 5])
            pltpu.sync_copy(rows[jl % 5], acc.at[didx.at[jl % 10]], add=True)
        plsc.subcore_barrier()

        pltpu.async_copy(acc.at[pl.ds(rbase, RC)], rbufA, semG0)
        for rj in range(NCH_R):
            rb = bufs[rj % 2]
            row0 = rbase + rj * RC
            pltpu.make_async_copy(acc.at[pl.ds(row0, RC)], rb, semG0).wait()
            if rj + 1 < NCH_R:
                pltpu.async_copy(
                    acc.at[pl.ds(row0 + RC, RC)], bufs[(rj + 1) % 2], semG0)

            def _out_body(r, _, rj=rj, rb=rb):
                sv = dinv[rj * RC + r, :]
                for k in range(DH // 16):
                    rb[r, pl.ds(k * 16, 16)] = (
                        rb[r, pl.ds(k * 16, 16)] * sv + bvec[pl.ds(k * 16, 16)])
                return _
            lax.fori_loop(0, RC, _out_body, None)
            for cc, outp in ((0, outT), (1, outS)):
                @pl.when((c == cc) & (row0 + RC <= N))
                def _(outp=outp, row0=row0, rb=rb, p=p):
                    pltpu.sync_copy(
                        rb, outp.at[pl.ds(row0, RC), pl.ds(p * DH, DH)])

                @pl.when((c == cc) & (row0 < N) & (row0 + RC > N))
                def _(outp=outp, row0=row0, rb=rb, p=p):
                    pltpu.sync_copy(
                        rb.at[pl.ds(0, N % RC)],
                        outp.at[pl.ds(row0, N % RC), pl.ds(p * DH, DH)])

            @pl.when(c == 0)
            def _(row0=row0, rb=rb):
                for k in range(EC // 16):
                    rv = row0 + k * 16 + lax.iota(jnp.int32, 16)
                    didx[0, pl.ds(k * 16, 16)] = (rv * 5243) >> 19

                def _q_body(r, _):
                    for k in range(DH // 16):
                        rows1[r, pl.ds(k * 16, 16)] = (
                            rb[r, pl.ds(k * 16, 16)] * (1.0 / (N // B)))
                    return _
                lax.fori_loop(0, RC, _q_body, None)
                pltpu.sync_copy(rows1, qacc.at[didx.at[0]], add=True)
        plsc.subcore_barrier()

        @pl.when((c == 0) & (s == 0))
        def _(p=p):
            pltpu.sync_copy(qacc, rows1.at[pl.ds(0, B + 4)])
            pltpu.sync_copy(rows1.at[pl.ds(0, B + 4)],
                            qout.at[pl.ds(0, B + 4), pl.ds(p * DH, DH)])
            if p == 0:
                def _zq_body(r, _):
                    for k in range(DH // 16):
                        rows1[r, pl.ds(k * 16, 16)] = jnp.zeros(
                            (16,), jnp.float32)
                    return _
                lax.fori_loop(0, B + 4, _zq_body, None)
                pltpu.sync_copy(rows1.at[pl.ds(0, B + 4)], qacc)


@jax.jit
def _sc_gcn(xw0, xw1, eir, bs):
    mesh = plsc.VectorSubcoreMesh(core_axis_name="c", subcore_axis_name="s")
    f = pl.kernel(
        _sc_gcn_body,
        out_type=[jax.ShapeDtypeStruct((N, D), jnp.float32),
                  jax.ShapeDtypeStruct((N, D), jnp.float32),
                  jax.ShapeDtypeStruct((B + 4, D), jnp.float32),
                  jax.ShapeDtypeStruct((2 * NPAD, DH), jnp.float32),
                  jax.ShapeDtypeStruct((2 * NPAD, DH), jnp.float32)],
        mesh=mesh,
        compiler_params=pltpu.CompilerParams(use_tc_tiling_on_sc=False),
        scratch_types=[
            pltpu.VMEM_SHARED((NPAD, DH), jnp.float32),
            pltpu.VMEM_SHARED((NPAD, 16), jnp.float32),
            pltpu.VMEM_SHARED((B + 4, DH), jnp.float32),
            pltpu.VMEM((10, EC), jnp.int32),
            pltpu.VMEM((10, EC), jnp.int32),
            pltpu.VMEM((EC, DH), jnp.float32),
            pltpu.VMEM((EC, DH), jnp.float32),
            pltpu.VMEM((EC, DH), jnp.float32),
            pltpu.VMEM((EC, DH), jnp.float32),
            pltpu.VMEM((EC, DH), jnp.float32),
            pltpu.VMEM((EC, 16), jnp.float32),
            pltpu.VMEM((RT, 16), jnp.float32),
            pltpu.VMEM((RC, DH), jnp.float32),
            pltpu.VMEM((RC, DH), jnp.float32),
            pltpu.VMEM((DH,), jnp.float32),
            pltpu.SemaphoreType.DMA,
            pltpu.SemaphoreType.DMA,
            pltpu.SemaphoreType.DMA,
            pltpu.SemaphoreType.DMA,
            pltpu.SemaphoreType.DMA,
            pltpu.SemaphoreType.DMA,
            pltpu.SemaphoreType.DMA,
            pltpu.SemaphoreType.DMA,
            pltpu.SemaphoreType.DMA,
            pltpu.SemaphoreType.DMA,
            pltpu.SemaphoreType.DMA,
            pltpu.SemaphoreType.DMA,
            pltpu.SemaphoreType.DMA,
        ],
    )
    return f(xw0, xw1, eir, bs)


def _mm_body(x_ref, w_ref, o0_ref, o1_ref):
    r = jnp.dot(x_ref[0], w_ref[0], preferred_element_type=jnp.float32)
    o0_ref[0] = r[:, :DH]
    o1_ref[0] = r[:, DH:]


@jax.jit
def _tc_matmul(xs, Ws):
    BM = 1280
    return pl.pallas_call(
        _mm_body,
        grid=(2, NPAD // BM),
        in_specs=[pl.BlockSpec((1, BM, D), lambda b, i: (b, i, 0)),
                  pl.BlockSpec((1, D, D), lambda b, i: (b, 0, 0))],
        out_specs=[pl.BlockSpec((1, BM, DH), lambda b, i: (b, i, 0)),
                   pl.BlockSpec((1, BM, DH), lambda b, i: (b, i, 0))],
        out_shape=[jax.ShapeDtypeStruct((2, NPAD, DH), jnp.float32),
                   jax.ShapeDtypeStruct((2, NPAD, DH), jnp.float32)],
    )(xs, Ws)


def kernel(text_x, text_edge_index, text_batch, scene_x, scene_edge_index,
           scene_batch, W_text, b_text, W_scene, b_scene):
    Ws = jnp.stack([W_text, W_scene])
    bs = jnp.stack([b_text, b_scene])
    ei = jnp.stack([text_edge_index, scene_edge_index])
    ei_p = jnp.concatenate(
        [ei, jnp.full((2, 2, EPAD - E), NPAD - 1, ei.dtype)], axis=-1)
    eir = ei_p.reshape(2, 2, 16, NCH_E, EC)

    xs = jnp.stack([text_x, scene_x])
    xw0, xw1 = _tc_matmul(xs, Ws)
    out_t, out_s, qpad, _, _ = _sc_gcn(xw0, xw1, eir, bs)
    question = qpad[:B]

    contextual_words = out_t.reshape(B, N // B, D)
    scene_graph_feats = out_s.reshape(B, N // B, D)
    return contextual_words, question, scene_graph_feats


# speedup vs baseline: 1.0296x; 1.0141x over previous
import jax
import jax.numpy as jnp
from jax import lax
from jax.experimental import pallas as pl
from jax.experimental.pallas import tpu as pltpu
from jax.experimental.pallas import tpu_sc as plsc

N = 10000
E = 320000
D = 128
B = 100

DH = D // 2
NPAD = 10240
RT = NPAD // 16
RC = 128
EC = 128
EPAD = 321536
ETP = EPAD // 16
NCH_E = ETP // EC
NCH_R = RT // RC


def _fast_rsqrt(d):
    xi = lax.bitcast_convert_type(d, jnp.int32)
    xi = jnp.int32(0x5F3759DF) - (xi >> 1)
    r = lax.bitcast_convert_type(xi, jnp.float32)
    r = r * (1.5 - 0.5 * d * r * r)
    r = r * (1.5 - 0.5 * d * r * r)
    r = r * (1.5 - 0.5 * d * r * r)
    return r


def _sc_gcn_body(xw0, xw1, eir, bs, outT, outS, qout, y0, y1, acc, degsp, qacc,
                 sidx, didx, rows0, rows1, rows2, rows3, rows4, ones,
                 dinv, rbufA, rbufB, bvec,
                 semG0, semG1, semG2, semG3, semG4,
                 semS0, semS1, semS2, semS3, semS4,
                 semD, semW, semI):
    c = lax.axis_index("c")
    s = lax.axis_index("s")
    boff = c * NPAD
    rows = (rows0, rows1, rows2, rows3, rows4)
    semG = (semG0, semG1, semG2, semG3, semG4)
    semS = (semS0, semS1, semS2, semS3, semS4)

    def _zero_body(r, _):
        dinv[r, :] = jnp.zeros((16,), jnp.float32)
        return _
    lax.fori_loop(0, RT, _zero_body, None)

    def _ones_body(r, _):
        ones[r, :] = jnp.ones((16,), jnp.float32)
        return _
    lax.fori_loop(0, EC, _ones_body, None)

    pltpu.sync_copy(dinv, degsp.at[pl.ds(RT * s, RT)])

    @pl.when((c == 0) & (s == 0))
    def _():
        def _zq_body(r, _):
            for k in range(DH // 16):
                rows0[r, pl.ds(k * 16, 16)] = jnp.zeros((16,), jnp.float32)
            return _
        lax.fori_loop(0, B + 4, _zq_body, None)
        pltpu.sync_copy(rows0.at[pl.ds(0, B + 4)], qacc)
    plsc.subcore_barrier()

    def _didx_load(j, sem):
        pltpu.async_copy(eir.at[c, 1, s, j], didx.at[j % 8], sem)

    def _didx_wait(j, sem):
        pltpu.make_async_copy(eir.at[c, 1, s, j], didx.at[j % 8], sem).wait()

    for j in range(4):
        _didx_load(j, semI)
    for j in range(4):
        _didx_wait(j, semI)
    for j in range(4, 8):
        _didx_load(j, semI)

    def _deg_body(q, _):
        j0 = 4 * q
        for b in range(4):
            pltpu.async_copy(ones, degsp.at[didx.at[(j0 + b) % 8]], semD,
                             add=True)
        for b in range(4):
            @pl.when(j0 + 4 + b < NCH_E)
            def _(b=b):
                _didx_wait(j0 + 4 + b, semI)
        for b in range(4):
            pltpu.make_async_copy(ones, degsp.at[didx.at[(j0 + b) % 8]],
                                  semD).wait()
        for b in range(4):
            @pl.when(j0 + 8 + b < NCH_E)
            def _(b=b):
                _didx_load(j0 + 8 + b, semI)
        return _
    lax.fori_loop(0, (NCH_E - 1) // 4, _deg_body, None)
    jl = NCH_E - 1
    pltpu.sync_copy(ones, degsp.at[didx.at[jl % 8]], add=True)
    plsc.subcore_barrier()

    pltpu.sync_copy(degsp.at[pl.ds(RT * s, RT)], dinv)

    def _rs_body(r, _):
        d = dinv[r, :] + 1.0
        dinv[r, :] = _fast_rsqrt(d)
        return _
    lax.fori_loop(0, RT, _rs_body, None)

    rbase = s * RT
    bufs = (rbufA, rbufB)
    for p, xwp, yp in ((0, xw0, y0), (1, xw1, y1)):
        pltpu.sync_copy(bs.at[c, pl.ds(p * DH, DH)], bvec)

        pltpu.async_copy(xwp.at[c, pl.ds(rbase, RC)], rbufA, semG0)
        for rj in range(NCH_R):
            rb = bufs[rj % 2]
            row0 = rbase + rj * RC
            pltpu.make_async_copy(xwp.at[c, pl.ds(row0, RC)], rb, semG0).wait()
            if rj + 1 < NCH_R:
                pltpu.async_copy(
                    xwp.at[c, pl.ds(row0 + RC, RC)], bufs[(rj + 1) % 2], semG0)

            def _scale_body(r, _, rj=rj, rb=rb):
                sv = dinv[rj * RC + r, :]
                for k in range(DH // 16):
                    rb[r, pl.ds(k * 16, 16)] = rb[r, pl.ds(k * 16, 16)] * sv
                return _
            lax.fori_loop(0, RC, _scale_body, None)
            pltpu.async_copy(rb, yp.at[pl.ds(c * NPAD + row0, RC)], semW)
            pltpu.sync_copy(rb, acc.at[pl.ds(row0, RC)])
            pltpu.make_async_copy(rb, yp.at[pl.ds(c * NPAD + row0, RC)], semW).wait()
        plsc.subcore_barrier()

        def _eidx_load(j, sem, yp=yp):
            pltpu.async_copy(eir.at[c, 0, s, j], sidx.at[j % 10], sem)
            pltpu.async_copy(eir.at[c, 1, s, j], didx.at[j % 10], sem)

        def _idx_wait(j, sem, yp=yp):
            pltpu.make_async_copy(eir.at[c, 0, s, j], sidx.at[j % 10], sem).wait()
            pltpu.make_async_copy(eir.at[c, 1, s, j], didx.at[j % 10], sem).wait()

        def _gather(j, rb, sem, yp=yp):
            for k in range(EC // 16):
                sidx[j % 10, pl.ds(k * 16, 16)] = (
                    sidx[j % 10, pl.ds(k * 16, 16)] + boff)
            pltpu.async_copy(yp.at[sidx.at[j % 10]], rb, sem)

        def _gather_wait(j, rb, sem, yp=yp):
            pltpu.make_async_copy(yp.at[sidx.at[j % 10]], rb, sem).wait()

        def _scat(j, rb, sem):
            pltpu.async_copy(rb, acc.at[didx.at[j % 10]], sem, add=True)

        def _scat_wait(j, rb, sem):
            pltpu.make_async_copy(rb, acc.at[didx.at[j % 10]], sem).wait()

        for j in range(5):
            _eidx_load(j, semI)
        for j in range(5):
            _idx_wait(j, semI)
        for b in range(5):
            _gather(b, rows[b], semG[b])
        for j in range(5, 10):
            _eidx_load(j, semI)

        def _edge_body(q, _, yp=yp):
            j0 = 5 * q
            for b in range(5):
                _gather_wait(j0 + b, rows[b], semG[b])
                _scat(j0 + b, rows[b], semS[b])
            for b in range(5):
                @pl.when(j0 + 5 + b < NCH_E)
                def _(b=b):
                    _idx_wait(j0 + 5 + b, semI)
            for b in range(5):
                _scat_wait(j0 + b, rows[b], semS[b])

                @pl.when(j0 + 5 + b < NCH_E)
                def _(b=b):
                    _gather(j0 + 5 + b, rows[b], semG[b])
            for b in range(5):
                @pl.when(j0 + 10 + b < NCH_E)
                def _(b=b):
                    _eidx_load(j0 + 10 + b, semI)
            return _
        lax.fori_loop(0, (NCH_E - 2) // 5, _edge_body, None)
        for jl in (NCH_E - 2, NCH_E - 1):
            _gather_wait(jl, rows[jl % 5], semG[jl % 5])
            pltpu.sync_copy(rows[jl % 5], acc.at[didx.at[jl % 10]], add=True)
        plsc.subcore_barrier()

        pltpu.async_copy(acc.at[pl.ds(rbase, RC)], rbufA, semG0)
        for rj in range(NCH_R):
            rb = bufs[rj % 2]
            row0 = rbase + rj * RC
            pltpu.make_async_copy(acc.at[pl.ds(row0, RC)], rb, semG0).wait()
            if rj + 1 < NCH_R:
                pltpu.async_copy(
                    acc.at[pl.ds(row0 + RC, RC)], bufs[(rj + 1) % 2], semG0)

            def _out_body(r, _, rj=rj, rb=rb):
                sv = dinv[rj * RC + r, :]
                for k in range(DH // 16):
                    rb[r, pl.ds(k * 16, 16)] = (
                        rb[r, pl.ds(k * 16, 16)] * sv + bvec[pl.ds(k * 16, 16)])
                return _
            lax.fori_loop(0, RC, _out_body, None)
            for cc, outp in ((0, outT), (1, outS)):
                @pl.when((c == cc) & (row0 + RC <= N))
                def _(outp=outp, row0=row0, rb=rb, p=p):
                    pltpu.sync_copy(
                        rb, outp.at[pl.ds(row0, RC), pl.ds(p * DH, DH)])

                @pl.when((c == cc) & (row0 < N) & (row0 + RC > N))
                def _(outp=outp, row0=row0, rb=rb, p=p):
                    pltpu.sync_copy(
                        rb.at[pl.ds(0, N % RC)],
                        outp.at[pl.ds(row0, N % RC), pl.ds(p * DH, DH)])

            @pl.when(c == 0)
            def _(row0=row0, rb=rb):
                for k in range(EC // 16):
                    rv = row0 + k * 16 + lax.iota(jnp.int32, 16)
                    didx[0, pl.ds(k * 16, 16)] = (rv * 5243) >> 19

                def _q_body(r, _):
                    for k in range(DH // 16):
                        rows1[r, pl.ds(k * 16, 16)] = (
                            rb[r, pl.ds(k * 16, 16)] * (1.0 / (N // B)))
                    return _
                lax.fori_loop(0, RC, _q_body, None)
                pltpu.sync_copy(rows1, qacc.at[didx.at[0]], add=True)
        plsc.subcore_barrier()

        @pl.when((c == 0) & (s == 0))
        def _(p=p):
            pltpu.sync_copy(qacc, rows1.at[pl.ds(0, B + 4)])
            pltpu.sync_copy(rows1.at[pl.ds(0, B + 4)],
                            qout.at[pl.ds(0, B + 4), pl.ds(p * DH, DH)])
            if p == 0:
                def _zq_body(r, _):
                    for k in range(DH // 16):
                        rows1[r, pl.ds(k * 16, 16)] = jnp.zeros(
                            (16,), jnp.float32)
                    return _
                lax.fori_loop(0, B + 4, _zq_body, None)
                pltpu.sync_copy(rows1.at[pl.ds(0, B + 4)], qacc)


@jax.jit
def _sc_gcn(xw0, xw1, eir, bs):
    mesh = plsc.VectorSubcoreMesh(core_axis_name="c", subcore_axis_name="s")
    f = pl.kernel(
        _sc_gcn_body,
        out_type=[jax.ShapeDtypeStruct((N, D), jnp.float32),
                  jax.ShapeDtypeStruct((N, D), jnp.float32),
                  jax.ShapeDtypeStruct((B + 4, D), jnp.float32),
                  jax.ShapeDtypeStruct((2 * NPAD, DH), jnp.float32),
                  jax.ShapeDtypeStruct((2 * NPAD, DH), jnp.float32)],
        mesh=mesh,
        compiler_params=pltpu.CompilerParams(use_tc_tiling_on_sc=False),
        scratch_types=[
            pltpu.VMEM_SHARED((NPAD, DH), jnp.float32),
            pltpu.VMEM_SHARED((NPAD, 16), jnp.float32),
            pltpu.VMEM_SHARED((B + 4, DH), jnp.float32),
            pltpu.VMEM((10, EC), jnp.int32),
            pltpu.VMEM((10, EC), jnp.int32),
            pltpu.VMEM((EC, DH), jnp.float32),
            pltpu.VMEM((EC, DH), jnp.float32),
            pltpu.VMEM((EC, DH), jnp.float32),
            pltpu.VMEM((EC, DH), jnp.float32),
            pltpu.VMEM((EC, DH), jnp.float32),
            pltpu.VMEM((EC, 16), jnp.float32),
            pltpu.VMEM((RT, 16), jnp.float32),
            pltpu.VMEM((RC, DH), jnp.float32),
            pltpu.VMEM((RC, DH), jnp.float32),
            pltpu.VMEM((DH,), jnp.float32),
            pltpu.SemaphoreType.DMA,
            pltpu.SemaphoreType.DMA,
            pltpu.SemaphoreType.DMA,
            pltpu.SemaphoreType.DMA,
            pltpu.SemaphoreType.DMA,
            pltpu.SemaphoreType.DMA,
            pltpu.SemaphoreType.DMA,
            pltpu.SemaphoreType.DMA,
            pltpu.SemaphoreType.DMA,
            pltpu.SemaphoreType.DMA,
            pltpu.SemaphoreType.DMA,
            pltpu.SemaphoreType.DMA,
            pltpu.SemaphoreType.DMA,
        ],
    )
    return f(xw0, xw1, eir, bs)


def _mm_body(tx_ref, sx_ref, w_ref, o0_ref, o1_ref):
    b = pl.program_id(0)

    @pl.when(b == 0)
    def _():
        r = jnp.dot(tx_ref[...], w_ref[0], preferred_element_type=jnp.float32)
        o0_ref[0] = r[:, :DH]
        o1_ref[0] = r[:, DH:]

    @pl.when(b == 1)
    def _():
        r = jnp.dot(sx_ref[...], w_ref[0], preferred_element_type=jnp.float32)
        o0_ref[0] = r[:, :DH]
        o1_ref[0] = r[:, DH:]


@jax.jit
def _tc_matmul(tx, sx, Ws):
    BM = 1280
    return pl.pallas_call(
        _mm_body,
        grid=(2, NPAD // BM),
        in_specs=[pl.BlockSpec((BM, D), lambda b, i: (i, 0)),
                  pl.BlockSpec((BM, D), lambda b, i: (i, 0)),
                  pl.BlockSpec((1, D, D), lambda b, i: (b, 0, 0))],
        out_specs=[pl.BlockSpec((1, BM, DH), lambda b, i: (b, i, 0)),
                   pl.BlockSpec((1, BM, DH), lambda b, i: (b, i, 0))],
        out_shape=[jax.ShapeDtypeStruct((2, NPAD, DH), jnp.float32),
                   jax.ShapeDtypeStruct((2, NPAD, DH), jnp.float32)],
    )(tx, sx, Ws)


def kernel(text_x, text_edge_index, text_batch, scene_x, scene_edge_index,
           scene_batch, W_text, b_text, W_scene, b_scene):
    Ws = jnp.stack([W_text, W_scene])
    bs = jnp.stack([b_text, b_scene])
    ei = jnp.stack([text_edge_index, scene_edge_index])
    ei_p = jnp.concatenate(
        [ei, jnp.full((2, 2, EPAD - E), NPAD - 1, ei.dtype)], axis=-1)
    eir = ei_p.reshape(2, 2, 16, NCH_E, EC)

    xw0, xw1 = _tc_matmul(text_x, scene_x, Ws)
    out_t, out_s, qpad, _, _ = _sc_gcn(xw0, xw1, eir, bs)
    question = qpad[:B]

    contextual_words = out_t.reshape(B, N // B, D)
    scene_graph_feats = out_s.reshape(B, N // B, D)
    return contextual_words, question, scene_graph_feats
